# Initial kernel scaffold; baseline (speedup 1.0000x reference)
import jax
import jax.numpy as jnp
from jax.experimental import pallas as pl


def kernel(x, G, W1, b1, W2, att, b2, W3, b3):
    raise NotImplementedError("write your pallas kernel here")


# speedup vs baseline: 11.7647x; 11.7647x over previous
import jax
import jax.numpy as jnp
from jax.experimental import pallas as pl

N = 40
TXT_FEAT_LEN = 1386
K1PAD = 1408
HIDDEN = 4096
HEADS = 8
CODE_LEN = 64
NEG_SLOPE = 0.2

A_TILE = 512
B_TILE = 1024


def _mask_degrees(G):
    Mf = (G != -1.5).astype(jnp.float32)
    D = jnp.sum(Mf, axis=1)
    B = jnp.sum(Mf, axis=0)
    Dinv = jnp.where(D > 0.0, 1.0 / D, 0.0)
    Binv = jnp.where(B > 0.0, 1.0 / B, 0.0)
    return Mf, B, Dinv, Binv


def _mm(a, b):
    return jax.lax.dot_general(a, b, (((1,), (0,)), ((), ())),
                               preferred_element_type=jnp.float32)


def _mtm(a, b):
    return jax.lax.dot_general(a, b, (((0,), (0,)), ((), ())),
                               preferred_element_type=jnp.float32)


def _feat_kernel(x_ref, w1_ref, g_ref, b1_ref, feat_ref, hattr_ref):
    Mf, B, Dinv, Binv = _mask_degrees(g_ref[...])
    xw = _mm(x_ref[...], w1_ref[...])
    ef = Binv[:, None] * _mtm(Mf, xw)
    feat = jnp.maximum(Dinv[:, None] * _mm(Mf, ef) + b1_ref[...], 0.0)
    feat_ref[...] = feat
    hattr_ref[...] = _mtm(Mf, feat) / B[:, None]


def _gemm_kernel(fc_ref, w2_ref, out_ref):
    out_ref[...] = _mm(fc_ref[...], w2_ref[...])


def _att_kernel(xw_ref, g_ref, att1_ref, att2_ref, feat_ref, b2_ref,
                w3_ref, b3_ref, hid_ref, code_ref):
    Mb = g_ref[...] != -1.5
    Mf = Mb.astype(jnp.float32)
    B = jnp.sum(Mf, axis=0)
    D = jnp.sum(Mf, axis=1)
    Dinv = jnp.where(D > 0.0, 1.0 / D, 0.0)
    Binv = jnp.where(B > 0.0, 1.0 / B, 0.0)

    xw = xw_ref[...]
    acc = None
    for h in range(HEADS):
        xh_all = xw[:, h * HIDDEN:(h + 1) * HIDDEN]
        xh = xh_all[:N]
        eh = xh_all[N:]
        ax = jnp.sum(xh * att1_ref[h, :], axis=1)
        ae = jnp.sum(eh * att2_ref[h, :], axis=1)
        al = ax[:, None] + ae[None, :]
        al = jnp.where(al >= 0.0, al, NEG_SLOPE * al)
        amax = jnp.max(jnp.where(Mb, al, -jnp.inf), axis=1)
        amax = jnp.where(amax > -jnp.inf, amax, 0.0)
        aexp = jnp.where(Mb, jnp.exp(al - amax[:, None]), 0.0)
        asum = jnp.sum(aexp, axis=1)
        A = aexp / (asum[:, None] + 1e-16)
        ef = Binv[:, None] * _mtm(A, xh)
        oh = Dinv[:, None] * _mm(A, ef)
        acc = oh if acc is None else acc + oh
    hid_att = acc * (1.0 / HEADS) + b2_ref[...]
    hid_cat = feat_ref[...] + hid_att

    xw3 = _mm(hid_cat, w3_ref[...])
    ef3 = Binv[:, None] * _mtm(Mf, xw3)
    hid = Dinv[:, None] * _mm(Mf, ef3) + b3_ref[...]
    hid_ref[...] = hid
    code_ref[...] = jnp.tanh(hid)


def kernel(x, G, W1, b1, W2, att, b2, W3, b3):
    xp = jnp.pad(x, ((0, 0), (0, K1PAD - TXT_FEAT_LEN)))
    W1p = jnp.pad(W1, ((0, K1PAD - TXT_FEAT_LEN), (0, 0)))
    b1r = b1.reshape(1, HIDDEN)

    n_a = HIDDEN // A_TILE
    feat, hattr = pl.pallas_call(
        _feat_kernel,
        grid=(n_a,),
        in_specs=[
            pl.BlockSpec((N, K1PAD), lambda t: (0, 0)),
            pl.BlockSpec((K1PAD, A_TILE), lambda t: (0, t)),
            pl.BlockSpec((N, N), lambda t: (0, 0)),
            pl.BlockSpec((1, A_TILE), lambda t: (0, t)),
        ],
        out_specs=[
            pl.BlockSpec((N, A_TILE), lambda t: (0, t)),
            pl.BlockSpec((N, A_TILE), lambda t: (0, t)),
        ],
        out_shape=[
            jax.ShapeDtypeStruct((N, HIDDEN), jnp.float32),
            jax.ShapeDtypeStruct((N, HIDDEN), jnp.float32),
        ],
    )(xp, W1p, G, b1r)

    fc = jnp.concatenate([feat, hattr], axis=0)
    n_b = (HEADS * HIDDEN) // B_TILE
    xw = pl.pallas_call(
        _gemm_kernel,
        grid=(n_b,),
        in_specs=[
            pl.BlockSpec((2 * N, HIDDEN), lambda t: (0, 0)),
            pl.BlockSpec((HIDDEN, B_TILE), lambda t: (0, t)),
        ],
        out_specs=pl.BlockSpec((2 * N, B_TILE), lambda t: (0, t)),
        out_shape=jax.ShapeDtypeStruct((2 * N, HEADS * HIDDEN), jnp.float32),
    )(fc, W2)

    att1 = att[0, :, :HIDDEN]
    att2 = att[0, :, HIDDEN:]
    hid, code = pl.pallas_call(
        _att_kernel,
        out_shape=[
            jax.ShapeDtypeStruct((N, CODE_LEN), jnp.float32),
            jax.ShapeDtypeStruct((N, CODE_LEN), jnp.float32),
        ],
    )(xw, G, att1, att2, feat, b2.reshape(1, HIDDEN), W3, b3.reshape(1, CODE_LEN))

    return (feat, hid, code)


# speedup vs baseline: 12.8384x; 1.0913x over previous
import jax
import jax.numpy as jnp
from jax.experimental import pallas as pl

N = 40
TXT_FEAT_LEN = 1386
K1PAD = 1408
HIDDEN = 4096
HEADS = 8
CODE_LEN = 64
NEG_SLOPE = 0.2

A_TILE = 512
B_TILE = 1024


def _mask_degrees(G):
    Mf = (G != -1.5).astype(jnp.float32)
    D = jnp.sum(Mf, axis=1)
    B = jnp.sum(Mf, axis=0)
    Dinv = jnp.where(D > 0.0, 1.0 / D, 0.0)
    Binv = jnp.where(B > 0.0, 1.0 / B, 0.0)
    return Mf, B, Dinv, Binv


def _mm(a, b):
    return jax.lax.dot_general(a, b, (((1,), (0,)), ((), ())),
                               preferred_element_type=jnp.float32)


def _mtm(a, b):
    return jax.lax.dot_general(a, b, (((0,), (0,)), ((), ())),
                               preferred_element_type=jnp.float32)


def _feat_kernel(x_ref, w1_ref, g_ref, b1_ref, feat_ref, hattr_ref):
    Mf, B, Dinv, Binv = _mask_degrees(g_ref[...])
    xw = _mm(x_ref[...], w1_ref[...])
    ef = Binv[:, None] * _mtm(Mf, xw)
    feat = jnp.maximum(Dinv[:, None] * _mm(Mf, ef) + b1_ref[...], 0.0)
    feat_ref[...] = feat
    hattr_ref[...] = _mtm(Mf, feat) / B[:, None]


def _gemm_kernel(fc_ref, w2_ref, out_ref):
    out_ref[...] = _mm(fc_ref[...], w2_ref[...])


def _att_kernel(xw_ref, g_ref, att1_ref, att2_ref, feat_ref, b2_ref,
                w3_ref, b3_ref, hid_ref, code_ref):
    Mb = g_ref[...] != -1.5
    Mf = Mb.astype(jnp.float32)
    B = jnp.sum(Mf, axis=0)
    D = jnp.sum(Mf, axis=1)
    Dinv = jnp.where(D > 0.0, 1.0 / D, 0.0)
    Binv = jnp.where(B > 0.0, 1.0 / B, 0.0)

    xw = xw_ref[...]
    acc = None
    for h in range(HEADS):
        xh_all = xw[:, h * HIDDEN:(h + 1) * HIDDEN]
        xh = xh_all[:N]
        eh = xh_all[N:]
        ax = jnp.sum(xh * att1_ref[h, :], axis=1)
        ae = jnp.sum(eh * att2_ref[h, :], axis=1)
        al = ax[:, None] + ae[None, :]
        al = jnp.where(al >= 0.0, al, NEG_SLOPE * al)
        amax = jnp.max(jnp.where(Mb, al, -jnp.inf), axis=1)
        amax = jnp.where(amax > -jnp.inf, amax, 0.0)
        aexp = jnp.where(Mb, jnp.exp(al - amax[:, None]), 0.0)
        asum = jnp.sum(aexp, axis=1)
        A = aexp / (asum[:, None] + 1e-16)
        ef = Binv[:, None] * _mtm(A, xh)
        oh = Dinv[:, None] * _mm(A, ef)
        acc = oh if acc is None else acc + oh
    hid_att = acc * (1.0 / HEADS) + b2_ref[...]
    hid_cat = feat_ref[...] + hid_att

    xw3 = _mm(hid_cat, w3_ref[...])
    ef3 = Binv[:, None] * _mtm(Mf, xw3)
    hid = Dinv[:, None] * _mm(Mf, ef3) + b3_ref[...]
    hid_ref[...] = hid
    code_ref[...] = jnp.tanh(hid)


def kernel(x, G, W1, b1, W2, att, b2, W3, b3):
    xp = x
    W1p = W1
    b1r = b1.reshape(1, HIDDEN)

    n_a = HIDDEN // A_TILE
    feat, hattr = pl.pallas_call(
        _feat_kernel,
        grid=(n_a,),
        in_specs=[
            pl.BlockSpec((N, TXT_FEAT_LEN), lambda t: (0, 0)),
            pl.BlockSpec((TXT_FEAT_LEN, A_TILE), lambda t: (0, t)),
            pl.BlockSpec((N, N), lambda t: (0, 0)),
            pl.BlockSpec((1, A_TILE), lambda t: (0, t)),
        ],
        out_specs=[
            pl.BlockSpec((N, A_TILE), lambda t: (0, t)),
            pl.BlockSpec((N, A_TILE), lambda t: (0, t)),
        ],
        out_shape=[
            jax.ShapeDtypeStruct((N, HIDDEN), jnp.float32),
            jax.ShapeDtypeStruct((N, HIDDEN), jnp.float32),
        ],
    )(xp, W1p, G, b1r)

    fc = jnp.concatenate([feat, hattr], axis=0)
    n_b = (HEADS * HIDDEN) // B_TILE
    xw = pl.pallas_call(
        _gemm_kernel,
        grid=(n_b,),
        in_specs=[
            pl.BlockSpec((2 * N, HIDDEN), lambda t: (0, 0)),
            pl.BlockSpec((HIDDEN, B_TILE), lambda t: (0, t)),
        ],
        out_specs=pl.BlockSpec((2 * N, B_TILE), lambda t: (0, t)),
        out_shape=jax.ShapeDtypeStruct((2 * N, HEADS * HIDDEN), jnp.float32),
    )(fc, W2)

    att1 = att[0, :, :HIDDEN]
    att2 = att[0, :, HIDDEN:]
    hid, code = pl.pallas_call(
        _att_kernel,
        out_shape=[
            jax.ShapeDtypeStruct((N, CODE_LEN), jnp.float32),
            jax.ShapeDtypeStruct((N, CODE_LEN), jnp.float32),
        ],
    )(xw, G, att1, att2, feat, b2.reshape(1, HIDDEN), W3, b3.reshape(1, CODE_LEN))

    return (feat, hid, code)


# speedup vs baseline: 13.2440x; 1.0316x over previous
import jax
import jax.numpy as jnp
from jax.experimental import pallas as pl
from jax.experimental.pallas import tpu as pltpu

N = 40
TXT_FEAT_LEN = 1386
HIDDEN = 4096
HEADS = 8
CODE_LEN = 64
NEG_SLOPE = 0.2

A_TILE = 512
B_TILE = 1024
N_A = HIDDEN // A_TILE
N_B = (HEADS * HIDDEN) // B_TILE
SUBS = HIDDEN // B_TILE


def _mask_degrees(G):
    Mf = (G != -1.5).astype(jnp.float32)
    D = jnp.sum(Mf, axis=1)
    B = jnp.sum(Mf, axis=0)
    Dinv = jnp.where(D > 0.0, 1.0 / D, 0.0)
    Binv = jnp.where(B > 0.0, 1.0 / B, 0.0)
    return Mf, B, Dinv, Binv


def _mm(a, b):
    return jax.lax.dot_general(a, b, (((1,), (0,)), ((), ())),
                               preferred_element_type=jnp.float32)


def _mtm(a, b):
    return jax.lax.dot_general(a, b, (((0,), (0,)), ((), ())),
                               preferred_element_type=jnp.float32)


def _fused_kernel(x_ref, w1_ref, g_ref, b1_ref, w2_ref, att1_ref, att2_ref,
                  b2_ref, w3_ref, b3_ref, feat_ref, hid_ref, code_ref,
                  fc_s, xw_s):
    t = pl.program_id(0)

    @pl.when(t < N_A)
    def _phase_a():
        Mf, B, Dinv, Binv = _mask_degrees(g_ref[...])
        xw1 = _mm(x_ref[...], w1_ref[...])
        ef = Binv[:, None] * _mtm(Mf, xw1)
        feat = jnp.maximum(Dinv[:, None] * _mm(Mf, ef) + b1_ref[...], 0.0)
        feat_ref[...] = feat
        hattr = _mtm(Mf, feat) / B[:, None]
        fc_s[t] = jnp.concatenate([feat, hattr], axis=0)

    @pl.when((t >= N_A) & (t < N_A + N_B))
    def _phase_b():
        acc = jnp.zeros((2 * N, B_TILE), jnp.float32)
        for k in range(N_A):
            acc += _mm(fc_s[k], w2_ref[k * A_TILE:(k + 1) * A_TILE, :])
        xw_s[t - N_A] = acc

    @pl.when(t == N_A + N_B)
    def _phase_c():
        Mb = g_ref[...] != -1.5
        Mf, B, Dinv, Binv = _mask_degrees(g_ref[...])

        As = []
        for h in range(HEADS):
            ax = jnp.zeros((N,), jnp.float32)
            ae = jnp.zeros((N,), jnp.float32)
            for s in range(SUBS):
                blk = xw_s[SUBS * h + s]
                a1 = att1_ref[h, s * B_TILE:(s + 1) * B_TILE]
                a2 = att2_ref[h, s * B_TILE:(s + 1) * B_TILE]
                ax += jnp.sum(blk[:N] * a1, axis=1)
                ae += jnp.sum(blk[N:] * a2, axis=1)
            al = ax[:, None] + ae[None, :]
            al = jnp.where(al >= 0.0, al, NEG_SLOPE * al)
            amax = jnp.max(jnp.where(Mb, al, -jnp.inf), axis=1)
            amax = jnp.where(amax > -jnp.inf, amax, 0.0)
            aexp = jnp.where(Mb, jnp.exp(al - amax[:, None]), 0.0)
            asum = jnp.sum(aexp, axis=1)
            As.append(aexp / (asum[:, None] + 1e-16))

        xw3 = jnp.zeros((N, CODE_LEN), jnp.float32)
        rpt = B_TILE // A_TILE
        for s in range(SUBS):
            acc = jnp.zeros((N, B_TILE), jnp.float32)
            for h in range(HEADS):
                xh = xw_s[SUBS * h + s][:N]
                ef = Binv[:, None] * _mtm(As[h], xh)
                acc = acc + Dinv[:, None] * _mm(As[h], ef)
            featsub = jnp.concatenate(
                [fc_s[rpt * s + r][:N] for r in range(rpt)], axis=1)
            hcs = (featsub + acc * (1.0 / HEADS)
                   + b2_ref[0, s * B_TILE:(s + 1) * B_TILE])
            xw3 = xw3 + _mm(hcs, w3_ref[s * B_TILE:(s + 1) * B_TILE, :])
        ef3 = Binv[:, None] * _mtm(Mf, xw3)
        hid = Dinv[:, None] * _mm(Mf, ef3) + b3_ref[...]
        hid_ref[...] = hid
        code_ref[...] = jnp.tanh(hid)


def kernel(x, G, W1, b1, W2, att, b2, W3, b3):
    att1 = att[0, :, :HIDDEN]
    att2 = att[0, :, HIDDEN:]
    a_last = N_A - 1
    b_last = N_B - 1
    grid = N_A + N_B + 1
    feat, hid, code = pl.pallas_call(
        _fused_kernel,
        grid=(grid,),
        in_specs=[
            pl.BlockSpec((N, TXT_FEAT_LEN), lambda t: (0, 0)),
            pl.BlockSpec((TXT_FEAT_LEN, A_TILE),
                         lambda t: (0, jnp.minimum(t, a_last))),
            pl.BlockSpec((N, N), lambda t: (0, 0)),
            pl.BlockSpec((1, A_TILE), lambda t: (0, jnp.minimum(t, a_last))),
            pl.BlockSpec((HIDDEN, B_TILE),
                         lambda t: (0, jnp.clip(t - N_A, 0, b_last))),
            pl.BlockSpec((HEADS, HIDDEN), lambda t: (0, 0)),
            pl.BlockSpec((HEADS, HIDDEN), lambda t: (0, 0)),
            pl.BlockSpec((1, HIDDEN), lambda t: (0, 0)),
            pl.BlockSpec((HIDDEN, CODE_LEN), lambda t: (0, 0)),
            pl.BlockSpec((1, CODE_LEN), lambda t: (0, 0)),
        ],
        out_specs=[
            pl.BlockSpec((N, A_TILE), lambda t: (0, jnp.minimum(t, a_last))),
            pl.BlockSpec((N, CODE_LEN), lambda t: (0, 0)),
            pl.BlockSpec((N, CODE_LEN), lambda t: (0, 0)),
        ],
        out_shape=[
            jax.ShapeDtypeStruct((N, HIDDEN), jnp.float32),
            jax.ShapeDtypeStruct((N, CODE_LEN), jnp.float32),
            jax.ShapeDtypeStruct((N, CODE_LEN), jnp.float32),
        ],
        scratch_shapes=[
            pltpu.VMEM((N_A, 2 * N, A_TILE), jnp.float32),
            pltpu.VMEM((N_B, 2 * N, B_TILE), jnp.float32),
        ],
        compiler_params=pltpu.CompilerParams(
            dimension_semantics=("arbitrary",)),
    )(x, W1, G, b1.reshape(1, HIDDEN), W2, att1, att2,
      b2.reshape(1, HIDDEN), W3, b3.reshape(1, CODE_LEN))
    return (feat, hid, code)


# speedup vs baseline: 13.2469x; 1.0002x over previous
import jax
import jax.numpy as jnp
from jax.experimental import pallas as pl
from jax.experimental.pallas import tpu as pltpu

N = 40
TXT_FEAT_LEN = 1386
HIDDEN = 4096
HEADS = 8
CODE_LEN = 64
NEG_SLOPE = 0.2

A_TILE = 512
B_TILE = 1024
N_A = HIDDEN // A_TILE
N_B = (HEADS * HIDDEN) // B_TILE
SUBS = HIDDEN // B_TILE


def _mask_degrees(G):
    Mf = (G != -1.5).astype(jnp.float32)
    D = jnp.sum(Mf, axis=1)
    B = jnp.sum(Mf, axis=0)
    Dinv = jnp.where(D > 0.0, 1.0 / D, 0.0)
    Binv = jnp.where(B > 0.0, 1.0 / B, 0.0)
    return Mf, B, Dinv, Binv


def _mm(a, b):
    return jax.lax.dot_general(a, b, (((1,), (0,)), ((), ())),
                               preferred_element_type=jnp.float32)


def _mtm(a, b):
    return jax.lax.dot_general(a, b, (((0,), (0,)), ((), ())),
                               preferred_element_type=jnp.float32)


def _fused_kernel(x_ref, w1_ref, g_ref, b1_ref, w2_ref, att1_ref, att2_ref,
                  b2_ref, w3_ref, b3_ref, feat_ref, hid_ref, code_ref,
                  fc_s, xw_s):
    t = pl.program_id(0)

    @pl.when(t < N_A)
    def _phase_a():
        Mf, B, Dinv, Binv = _mask_degrees(g_ref[...])
        xw1 = _mm(x_ref[...], w1_ref[...])
        ef = Binv[:, None] * _mtm(Mf, xw1)
        feat = jnp.maximum(Dinv[:, None] * _mm(Mf, ef) + b1_ref[...], 0.0)
        feat_ref[...] = feat
        hattr = _mtm(Mf, feat) / B[:, None]
        fc_s[t] = jnp.concatenate([feat, hattr], axis=0)

    @pl.when((t >= N_A) & (t < N_A + N_B))
    def _phase_b():
        acc = jnp.zeros((2 * N, B_TILE), jnp.float32)
        for k in range(N_A):
            acc += _mm(fc_s[k].astype(jnp.bfloat16),
                       w2_ref[k * A_TILE:(k + 1) * A_TILE, :].astype(jnp.bfloat16))
        xw_s[t - N_A] = acc

    @pl.when(t == N_A + N_B)
    def _phase_c():
        Mb = g_ref[...] != -1.5
        Mf, B, Dinv, Binv = _mask_degrees(g_ref[...])

        As = []
        for h in range(HEADS):
            ax = jnp.zeros((N,), jnp.float32)
            ae = jnp.zeros((N,), jnp.float32)
            for s in range(SUBS):
                blk = xw_s[SUBS * h + s]
                a1 = att1_ref[h, s * B_TILE:(s + 1) * B_TILE]
                a2 = att2_ref[h, s * B_TILE:(s + 1) * B_TILE]
                ax += jnp.sum(blk[:N] * a1, axis=1)
                ae += jnp.sum(blk[N:] * a2, axis=1)
            al = ax[:, None] + ae[None, :]
            al = jnp.where(al >= 0.0, al, NEG_SLOPE * al)
            amax = jnp.max(jnp.where(Mb, al, -jnp.inf), axis=1)
            amax = jnp.where(amax > -jnp.inf, amax, 0.0)
            aexp = jnp.where(Mb, jnp.exp(al - amax[:, None]), 0.0)
            asum = jnp.sum(aexp, axis=1)
            As.append(aexp / (asum[:, None] + 1e-16))

        xw3 = jnp.zeros((N, CODE_LEN), jnp.float32)
        rpt = B_TILE // A_TILE
        for s in range(SUBS):
            acc = jnp.zeros((N, B_TILE), jnp.float32)
            for h in range(HEADS):
                xh = xw_s[SUBS * h + s][:N]
                ef = Binv[:, None] * _mtm(As[h], xh)
                acc = acc + Dinv[:, None] * _mm(As[h], ef)
            featsub = jnp.concatenate(
                [fc_s[rpt * s + r][:N] for r in range(rpt)], axis=1)
            hcs = (featsub + acc * (1.0 / HEADS)
                   + b2_ref[0, s * B_TILE:(s + 1) * B_TILE])
            xw3 = xw3 + _mm(hcs, w3_ref[s * B_TILE:(s + 1) * B_TILE, :])
        ef3 = Binv[:, None] * _mtm(Mf, xw3)
        hid = Dinv[:, None] * _mm(Mf, ef3) + b3_ref[...]
        hid_ref[...] = hid
        code_ref[...] = jnp.tanh(hid)


def kernel(x, G, W1, b1, W2, att, b2, W3, b3):
    att1 = att[0, :, :HIDDEN]
    att2 = att[0, :, HIDDEN:]
    a_last = N_A - 1
    b_last = N_B - 1
    grid = N_A + N_B + 1
    feat, hid, code = pl.pallas_call(
        _fused_kernel,
        grid=(grid,),
        in_specs=[
            pl.BlockSpec((N, TXT_FEAT_LEN), lambda t: (0, 0)),
            pl.BlockSpec((TXT_FEAT_LEN, A_TILE),
                         lambda t: (0, jnp.minimum(t, a_last))),
            pl.BlockSpec((N, N), lambda t: (0, 0)),
            pl.BlockSpec((1, A_TILE), lambda t: (0, jnp.minimum(t, a_last))),
            pl.BlockSpec((HIDDEN, B_TILE),
                         lambda t: (0, jnp.clip(t - N_A, 0, b_last))),
            pl.BlockSpec((HEADS, HIDDEN), lambda t: (0, 0)),
            pl.BlockSpec((HEADS, HIDDEN), lambda t: (0, 0)),
            pl.BlockSpec((1, HIDDEN), lambda t: (0, 0)),
            pl.BlockSpec((HIDDEN, CODE_LEN), lambda t: (0, 0)),
            pl.BlockSpec((1, CODE_LEN), lambda t: (0, 0)),
        ],
        out_specs=[
            pl.BlockSpec((N, A_TILE), lambda t: (0, jnp.minimum(t, a_last))),
            pl.BlockSpec((N, CODE_LEN), lambda t: (0, 0)),
            pl.BlockSpec((N, CODE_LEN), lambda t: (0, 0)),
        ],
        out_shape=[
            jax.ShapeDtypeStruct((N, HIDDEN), jnp.float32),
            jax.ShapeDtypeStruct((N, CODE_LEN), jnp.float32),
            jax.ShapeDtypeStruct((N, CODE_LEN), jnp.float32),
        ],
        scratch_shapes=[
            pltpu.VMEM((N_A, 2 * N, A_TILE), jnp.float32),
            pltpu.VMEM((N_B, 2 * N, B_TILE), jnp.float32),
        ],
        compiler_params=pltpu.CompilerParams(
            dimension_semantics=("arbitrary",)),
    )(x, W1, G, b1.reshape(1, HIDDEN), W2, att1, att2,
      b2.reshape(1, HIDDEN), W3, b3.reshape(1, CODE_LEN))
    return (feat, hid, code)
